# baseline (speedup 1.0000x reference)
import functools

import numpy as np
import jax
import jax.numpy as jnp
from jax import lax
from jax.experimental import pallas as pl
from jax.experimental.pallas import tpu as pltpu
from jax.experimental.pallas import tpu_sc as plsc

_ALPHA = 4
_NBUF = 3
_PARTS = 8


def _sc_body(nrows, band, w, nc, nw, fr, out, b0, b1, b2,
             l0, l1, l2, s0, s1, s2):
    bufs = (b0, b1, b2)
    lsem = (l0, l1, l2)
    ssem = (s0, s1, s2)
    nsteps = nrows * _PARTS // nw
    rows_per_step = nw // _PARTS
    wid = lax.axis_index("s") * nc + lax.axis_index("c")
    row_sel = lax.shift_right_logical(wid, 3)
    band_lo = (wid & (_PARTS - 1)) * band

    def mk_load(k):
        b = k % _NBUF
        row = jnp.int32(k * rows_per_step) + row_sel
        c = lax.shift_right_logical(row, 4)
        t = row & 15
        src_row = c * 64 + lax.shift_right_logical(t * 1101005, 18)
        return pltpu.make_async_copy(
            fr.at[src_row, pl.ds(band_lo, band), :], bufs[b], lsem[b])

    def mk_store(k):
        b = k % _NBUF
        dst_row = jnp.int32(k * rows_per_step) + row_sel
        return pltpu.make_async_copy(
            bufs[b], out.at[dst_row, pl.ds(band_lo, band), :], ssem[b])

    for k in range(min(_NBUF, nsteps)):
        mk_load(k).start()
    for k in range(nsteps):
        mk_load(k).wait()
        mk_store(k).start()
        if k + _NBUF < nsteps:
            mk_store(k).wait()
            mk_load(k + _NBUF).start()
    for k in range(max(nsteps - _NBUF, 0), nsteps):
        mk_store(k).wait()


def kernel(frames):
    C, T, H, W = frames.shape
    Ts = T // _ALPHA
    band = H // _PARTS

    info = plsc.get_sparse_core_info()
    nc, ns = info.num_cores, info.num_subcores
    nw = nc * ns

    mesh = plsc.VectorSubcoreMesh(core_axis_name="c", subcore_axis_name="s")
    sc_copy = pl.kernel(
        functools.partial(_sc_body, C * Ts, band, W, nc, nw),
        out_type=jax.ShapeDtypeStruct((C * Ts, H, W), frames.dtype),
        mesh=mesh,
        scratch_types=(
            [pltpu.VMEM((band, W), frames.dtype) for _ in range(_NBUF)]
            + [pltpu.SemaphoreType.DMA] * (2 * _NBUF)
        ),
        compiler_params=pltpu.CompilerParams(
            use_tc_tiling_on_sc=True,
            skip_device_barrier=True,
            disable_bounds_checks=True,
            disable_semaphore_checks=True,
        ),
    )
    slow = sc_copy(frames.reshape(C * T, H, W))
    return (slow.reshape(C, Ts, H, W), frames)


# speedup vs baseline: 1.0001x; 1.0001x over previous
import functools

import numpy as np
import jax
import jax.numpy as jnp
from jax import lax
from jax.experimental import pallas as pl
from jax.experimental.pallas import tpu as pltpu
from jax.experimental.pallas import tpu_sc as plsc

_ALPHA = 4
_NBUF = 3
_PARTS = 8


def _sc_body(nrows, band, w, nc, nw, fr, out, b0, b1, b2,
             l0, l1, l2, s0, s1, s2):
    bufs = (b0, b1, b2)
    lsem = (l0, l1, l2)
    ssem = (s0, s1, s2)
    nsteps = nrows * _PARTS // nw
    rows_per_step = nw // _PARTS
    wid = lax.axis_index("s") * nc + lax.axis_index("c")
    row_sel = lax.shift_right_logical(wid, 3)
    band_lo = (wid & (_PARTS - 1)) * band

    def mk_load(k):
        b = k % _NBUF
        row = jnp.int32(k * rows_per_step) + row_sel
        c = lax.shift_right_logical(row, 4)
        t = row & 15
        src_row = c * 64 + lax.shift_right_logical(t * 1101005, 18)
        return pltpu.make_async_copy(
            fr.at[src_row, pl.ds(band_lo, band), :], bufs[b], lsem[b])

    def mk_store(k):
        b = k % _NBUF
        dst_row = jnp.int32(k * rows_per_step) + row_sel
        return pltpu.make_async_copy(
            bufs[b], out.at[dst_row, pl.ds(band_lo, band), :], ssem[b])

    for k in range(min(_NBUF, nsteps)):
        mk_load(k).start()
    for k in range(nsteps):
        mk_load(k).wait()
        mk_store(k).start()
        if k + _NBUF < nsteps:
            mk_store(k).wait()
            mk_load(k + _NBUF).start()
    for k in range(max(nsteps - _NBUF, 0), nsteps):
        mk_store(k).wait()


def kernel(frames):
    C, T, H, W = frames.shape
    Ts = T // _ALPHA
    band = H // _PARTS

    info = plsc.get_sparse_core_info()
    nc, ns = info.num_cores, info.num_subcores
    nw = nc * ns

    mesh = plsc.VectorSubcoreMesh(core_axis_name="c", subcore_axis_name="s")
    sc_copy = pl.kernel(
        functools.partial(_sc_body, C * Ts, band, W, nc, nw),
        out_type=jax.ShapeDtypeStruct((C * Ts, H, W), frames.dtype),
        mesh=mesh,
        scratch_types=(
            [pltpu.VMEM((band, W), frames.dtype) for _ in range(_NBUF)]
            + [pltpu.SemaphoreType.DMA] * (2 * _NBUF)
        ),
        compiler_params=pltpu.CompilerParams(use_tc_tiling_on_sc=True),
    )
    slow = sc_copy(frames.reshape(C * T, H, W))
    return (slow.reshape(C, Ts, H, W), frames)
